# Initial kernel scaffold; baseline (speedup 1.0000x reference)
import jax
import jax.numpy as jnp
from jax.experimental import pallas as pl


def kernel(x, labels, centers):
    raise NotImplementedError("write your pallas kernel here")


# speedup vs baseline: 1.0616x; 1.0616x over previous
import functools

import jax
import jax.numpy as jnp
from jax import lax
from jax.experimental import pallas as pl
from jax.experimental.pallas import tpu as pltpu
from jax.experimental.pallas import tpu_sc as plsc

NUM_CLASS = 1000
D = 512
B = 16384
L = 16
NC, NS = 2, 16
NW = NC * NS
BPW = B // NW
CHUNK = 32
NCHUNK = BPW // CHUNK
FCHUNK = D // L


def _body(x_hbm, labels_hbm, centers_hbm, out_hbm,
          idx_v, xbuf, cbuf, accbuf, out_v, sem_x, sem_c):
    wid = lax.axis_index("s") * NC + lax.axis_index("c")
    base = wid * BPW
    pltpu.sync_copy(labels_hbm.at[pl.ds(base, BPW)], idx_v)
    lane = lax.iota(jnp.int32, L)
    inv = jnp.float32(1.0 / D)
    lo = jnp.float32(1e-12)
    hi = jnp.float32(1e12)

    for k in range(NCHUNK):
        rb = k * CHUNK
        cx = pltpu.async_copy(x_hbm.at[pl.ds(base + rb, CHUNK)], xbuf, sem_x)
        cc = pltpu.async_copy(centers_hbm.at[idx_v.at[pl.ds(rb, CHUNK)]],
                              cbuf, sem_c)
        cx.wait()
        cc.wait()

        def row_body(r, _):
            def feat_body(j, acc):
                xv = xbuf[r, pl.ds(j * L, L)]
                cv = cbuf[r, pl.ds(j * L, L)]
                d = xv - cv
                d2 = d * d
                d2 = jnp.minimum(jnp.maximum(d2, lo), hi)
                return acc + d2
            acc = lax.fori_loop(0, FCHUNK, feat_body,
                                jnp.zeros((L,), jnp.float32), unroll=4)
            accbuf[pl.ds(r * L, L)] = acc
            return 0
        lax.fori_loop(0, CHUNK, row_body, 0)

        for g in range(CHUNK // L):
            rows16 = (g * L + lane) * L
            tot = jnp.zeros((L,), jnp.float32)
            for c in range(L):
                tot = tot + plsc.load_gather(accbuf, [rows16 + c])
            out_v[pl.ds(rb + g * L, L)] = tot * inv

    pltpu.sync_copy(out_v, out_hbm.at[pl.ds(base, BPW)])


@functools.partial(jax.jit, static_argnames=())
def kernel(x, labels, centers):
    labels = labels.astype(jnp.int32)
    mesh = plsc.VectorSubcoreMesh(core_axis_name="c", subcore_axis_name="s")
    fn = pl.kernel(
        _body,
        out_type=jax.ShapeDtypeStruct((B,), jnp.float32),
        mesh=mesh,
        scratch_types=[
            pltpu.VMEM((BPW,), jnp.int32),
            pltpu.VMEM((CHUNK, D), jnp.float32),
            pltpu.VMEM((CHUNK, D), jnp.float32),
            pltpu.VMEM((CHUNK * L,), jnp.float32),
            pltpu.VMEM((BPW,), jnp.float32),
            pltpu.SemaphoreType.DMA,
            pltpu.SemaphoreType.DMA,
        ],
        compiler_params=pltpu.CompilerParams(needs_layout_passes=False),
    )
    return fn(x, labels, centers)


# speedup vs baseline: 1.4797x; 1.3938x over previous
import functools

import jax
import jax.numpy as jnp
from jax import lax
from jax.experimental import pallas as pl
from jax.experimental.pallas import tpu as pltpu
from jax.experimental.pallas import tpu_sc as plsc

NUM_CLASS = 1000
D = 512
B = 16384
L = 16
NC, NS = 2, 16
NW = NC * NS
BPW = B // NW
CHUNK = 32
NCHUNK = BPW // CHUNK
FCHUNK = D // L


def _body(x_hbm, labels_hbm, centers_hbm, out_hbm,
          idx_v, xbuf, cbuf, accbuf, out_v,
          sem_x0, sem_x1, sem_c0, sem_c1):
    wid = lax.axis_index("s") * NC + lax.axis_index("c")
    base = wid * BPW
    pltpu.sync_copy(labels_hbm.at[pl.ds(base, BPW)], idx_v)
    lane = lax.iota(jnp.int32, L)
    inv = jnp.float32(1.0 / D)
    lo = jnp.float32(1e-12)
    hi = jnp.float32(1e12)
    sems_x = (sem_x0, sem_x1)
    sems_c = (sem_c0, sem_c1)

    def start(k):
        slot = k % 2
        rb = k * CHUNK
        cx = pltpu.async_copy(x_hbm.at[pl.ds(base + rb, CHUNK)],
                              xbuf.at[slot], sems_x[slot])
        cc = pltpu.async_copy(centers_hbm.at[idx_v.at[pl.ds(rb, CHUNK)]],
                              cbuf.at[slot], sems_c[slot])
        return cx, cc

    pend = start(0)
    for k in range(NCHUNK):
        slot = k % 2
        rb = k * CHUNK
        cx, cc = pend
        cx.wait()
        cc.wait()
        if k + 1 < NCHUNK:
            pend = start(k + 1)

        def row_body(r, _):
            def feat_body(j, acc):
                xv = xbuf[slot, r, pl.ds(j * L, L)]
                cv = cbuf[slot, r, pl.ds(j * L, L)]
                d = xv - cv
                d2 = d * d
                d2 = jnp.minimum(jnp.maximum(d2, lo), hi)
                return acc + d2
            acc = lax.fori_loop(0, FCHUNK, feat_body,
                                jnp.zeros((L,), jnp.float32), unroll=4)
            accbuf[pl.ds(r * L, L)] = acc
            return 0
        lax.fori_loop(0, CHUNK, row_body, 0)

        for g in range(CHUNK // L):
            rows16 = (g * L + lane) * L
            tot = jnp.zeros((L,), jnp.float32)
            for c in range(L):
                tot = tot + plsc.load_gather(accbuf, [rows16 + c])
            out_v[pl.ds(rb + g * L, L)] = tot * inv

    pltpu.sync_copy(out_v, out_hbm.at[pl.ds(base, BPW)])


@functools.partial(jax.jit, static_argnames=())
def kernel(x, labels, centers):
    labels = labels.astype(jnp.int32)
    mesh = plsc.VectorSubcoreMesh(core_axis_name="c", subcore_axis_name="s")
    fn = pl.kernel(
        _body,
        out_type=jax.ShapeDtypeStruct((B,), jnp.float32),
        mesh=mesh,
        scratch_types=[
            pltpu.VMEM((BPW,), jnp.int32),
            pltpu.VMEM((2, CHUNK, D), jnp.float32),
            pltpu.VMEM((2, CHUNK, D), jnp.float32),
            pltpu.VMEM((CHUNK * L,), jnp.float32),
            pltpu.VMEM((BPW,), jnp.float32),
            pltpu.SemaphoreType.DMA,
            pltpu.SemaphoreType.DMA,
            pltpu.SemaphoreType.DMA,
            pltpu.SemaphoreType.DMA,
        ],
        compiler_params=pltpu.CompilerParams(needs_layout_passes=False),
    )
    return fn(x, labels, centers)
